# baseline (device time: 25851 ns/iter reference)
import jax
import jax.numpy as jnp
from jax import lax
from jax.experimental import pallas as pl
from jax.experimental.pallas import tpu as pltpu

N_DEV = 8


def kernel(x, dy, gamma):
    m, d = x.shape

    def body(x_ref, dy_ref, gamma_ref, out_ref, own_ref, gather_ref,
             send_sems, recv_sems):
        my = lax.axis_index("i")

        CH = 256
        dg = jnp.zeros((1, d), jnp.float32)
        db = jnp.zeros((1, d), jnp.float32)
        for c in range(m // CH):
            xs = x_ref[pl.ds(c * CH, CH), :]
            dys = dy_ref[pl.ds(c * CH, CH), :]
            mu = jnp.mean(xs, axis=1, keepdims=True)
            xc = xs - mu
            var = jnp.mean(xc * xc, axis=1, keepdims=True)
            xhat = xc * lax.rsqrt(var + 1e-5)
            dg = dg + jnp.sum(dys * xhat, axis=0, keepdims=True)
            db = db + jnp.sum(dys, axis=0, keepdims=True)
        own_ref[0:1, :] = dg
        own_ref[1:2, :] = db

        barrier_sem = pltpu.get_barrier_semaphore()
        for j in range(1, N_DEV):
            pl.semaphore_signal(
                barrier_sem, inc=1,
                device_id=(lax.rem(my + j, N_DEV),),
                device_id_type=pl.DeviceIdType.MESH,
            )
        pl.semaphore_wait(barrier_sem, N_DEV - 1)

        rdmas = []
        for j in range(1, N_DEV):
            r = pltpu.make_async_remote_copy(
                src_ref=own_ref,
                dst_ref=gather_ref.at[j - 1],
                send_sem=send_sems.at[j - 1],
                recv_sem=recv_sems.at[j - 1],
                device_id=(lax.rem(my + j, N_DEV),),
                device_id_type=pl.DeviceIdType.MESH,
            )
            r.start()
            rdmas.append(r)

        acc = own_ref[:, :]
        for j, r in enumerate(rdmas):
            r.wait_recv()
            acc = acc + gather_ref[j, :, :]
        for r in rdmas:
            r.wait_send()
        out_ref[:, :] = acc

    return pl.pallas_call(
        body,
        out_shape=jax.ShapeDtypeStruct((2, d), jnp.float32),
        in_specs=[
            pl.BlockSpec(memory_space=pltpu.VMEM),
            pl.BlockSpec(memory_space=pltpu.VMEM),
            pl.BlockSpec(memory_space=pltpu.VMEM),
        ],
        out_specs=pl.BlockSpec(memory_space=pltpu.VMEM),
        scratch_shapes=[
            pltpu.VMEM((2, d), jnp.float32),
            pltpu.VMEM((N_DEV - 1, 2, d), jnp.float32),
            pltpu.SemaphoreType.DMA((N_DEV - 1,)),
            pltpu.SemaphoreType.DMA((N_DEV - 1,)),
        ],
        compiler_params=pltpu.CompilerParams(collective_id=0),
    )(x, dy, gamma)
